# baseline (speedup 1.0000x reference)
import functools

import jax
import jax.numpy as jnp
from jax import lax
from jax.experimental import pallas as pl
from jax.experimental.pallas import tpu as pltpu
from jax.experimental.pallas import tpu_sc as plsc

N = 10000
E = 320000
H = 128
G = 128

NP = 10240
RB = 2560
NB = NP // RB

NSC = 2
NT = 16
C = 80
EPT = E // NT
NCHUNK = EPT // C
RPT = NP // NT
RCH = 32
F32 = jnp.float32


def _ln(v, scale, bias, eps=1e-5):
    mu = jnp.mean(v, axis=-1, keepdims=True)
    var = jnp.var(v, axis=-1, keepdims=True)
    return (v - mu) / jnp.sqrt(var + eps) * scale + bias


def _pre_body(xp_ref, wn_ref, bn_ref, lns_ref, lnb_ref, be_ref, ohT_ref,
              h_ref, hn_ref, hm_ref, cnt_ref):
    b = pl.program_id(0)
    h = jnp.dot(xp_ref[...], wn_ref[...], preferred_element_type=F32) + bn_ref[...]
    h_ref[...] = h
    hn = jnp.maximum(_ln(h, lns_ref[...], lnb_ref[...]), 0.0)
    hn_ref[...] = hn
    hm_ref[...] = hn + be_ref[...]

    @pl.when(b == 0)
    def _():
        cnt_ref[...] = jnp.zeros_like(cnt_ref)

    cnt_ref[...] += jnp.dot(ohT_ref[...], jnp.ones((RB, G), F32),
                            preferred_element_type=F32)

    @pl.when(b == NB - 1)
    def _():
        cnt_ref[...] = jnp.maximum(cnt_ref[...], 1.0)


def _midA_body(hcur_ref, hn_ref, acc_ref, ohT_ref,
               w1_ref, b1_ref, mls_ref, mlb_ref, w2_ref, b2_ref,
               hc2_ref, num_ref):
    b = pl.program_id(0)
    a = acc_ref[...]
    slabs = []
    for c in range(NSC):
        ex = a[c, :, 0:64]
        mex = a[c, :, 64:128]
        slabs.append(mex / jnp.maximum(ex, 1e-16))
    out = hn_ref[...] + jnp.concatenate(slabs, axis=1)
    y = jnp.dot(out, w1_ref[...], preferred_element_type=F32) + b1_ref[...]
    y = jnp.maximum(_ln(y, mls_ref[...], mlb_ref[...]), 0.0)
    y = jnp.dot(y, w2_ref[...], preferred_element_type=F32) + b2_ref[...]
    hc2 = hcur_ref[...] + y
    hc2_ref[...] = hc2

    @pl.when(b == 0)
    def _():
        num_ref[...] = jnp.zeros_like(num_ref)

    num_ref[...] += jnp.dot(ohT_ref[...], hc2, preferred_element_type=F32)


def _midB_body(hc2_ref, oh_ref, num_ref, cnt_ref, vnp_ref, lns_ref, lnb_ref,
               be_ref, vn_ref, hcn_ref, hnn_ref, hm_ref):
    vn_new = vnp_ref[...] + num_ref[...] / cnt_ref[...]
    vn_ref[...] = vn_new
    hcn = hc2_ref[...] + jnp.dot(oh_ref[...], vn_new, preferred_element_type=F32)
    hcn_ref[...] = hcn
    z = jnp.maximum(_ln(hcn, lns_ref[...], lnb_ref[...]), 0.0)
    hnn_ref[...] = z
    hm_ref[...] = z + be_ref[...]


def _lastB_body(hc2_ref, oh_ref, ohT_ref, num_ref, cnt_ref, vnp_ref,
                lns_ref, lnb_ref, out_ref):
    b = pl.program_id(0)
    vn_new = vnp_ref[...] + num_ref[...] / cnt_ref[...]
    hcn = hc2_ref[...] + jnp.dot(oh_ref[...], vn_new, preferred_element_type=F32)
    hf = jnp.maximum(_ln(hcn, lns_ref[...], lnb_ref[...]), 0.0)

    @pl.when(b == 0)
    def _():
        out_ref[...] = jnp.zeros_like(out_ref)

    out_ref[...] += jnp.dot(ohT_ref[...], hf, preferred_element_type=F32)

    @pl.when(b == NB - 1)
    def _():
        out_ref[...] = out_ref[...] / cnt_ref[...]


EB = 8000


def _ea_body(eattr_ref, we_ref, o_ref):
    y = jnp.dot(eattr_ref[...], we_ref[...], preferred_element_type=F32)
    o_ref[0] = y[:, 0:64]
    o_ref[1] = y[:, 64:128]


def _rowblk(shape):
    return pl.BlockSpec(shape, lambda b: (b, 0))


def _colblk(shape):
    return pl.BlockSpec(shape, lambda b: (0, b))


def _full(shape):
    return pl.BlockSpec(shape, lambda b: (0, 0))


def _scblk(shape):
    return pl.BlockSpec(shape, lambda b: (0, b, 0))


_EA = pl.pallas_call(
    _ea_body,
    grid=(E // EB,),
    in_specs=[_rowblk((EB, 8)), _full((8, H))],
    out_specs=pl.BlockSpec((2, EB, 64), lambda b: (0, b, 0)),
    out_shape=jax.ShapeDtypeStruct((2, E, 64), F32),
)

_PRE = pl.pallas_call(
    _pre_body,
    grid=(NB,),
    in_specs=[_rowblk((RB, 16)), _full((16, H)), _full((1, H)), _full((1, H)),
              _full((1, H)), _full((1, H)), _colblk((G, RB))],
    out_specs=[_rowblk((RB, H)), _rowblk((RB, H)), _rowblk((RB, H)),
               _full((G, G))],
    out_shape=[jax.ShapeDtypeStruct((NP, H), F32),
               jax.ShapeDtypeStruct((NP, H), F32),
               jax.ShapeDtypeStruct((NP, H), F32),
               jax.ShapeDtypeStruct((G, G), F32)],
)

_MIDA = pl.pallas_call(
    _midA_body,
    grid=(NB,),
    in_specs=[_rowblk((RB, H)), _rowblk((RB, H)),
              _scblk((NSC, RB, H)),
              _colblk((G, RB)), _full((H, 2 * H)),
              _full((1, 2 * H)), _full((1, 2 * H)), _full((1, 2 * H)),
              _full((2 * H, H)), _full((1, H))],
    out_specs=[_rowblk((RB, H)), _full((G, H))],
    out_shape=[jax.ShapeDtypeStruct((NP, H), F32),
               jax.ShapeDtypeStruct((G, H), F32)],
)

_MIDB = pl.pallas_call(
    _midB_body,
    grid=(NB,),
    in_specs=[_rowblk((RB, H)), _rowblk((RB, G)), _full((G, H)), _full((G, H)),
              _full((G, H)), _full((1, H)), _full((1, H)), _full((1, H))],
    out_specs=[_full((G, H)), _rowblk((RB, H)), _rowblk((RB, H)),
               _rowblk((RB, H))],
    out_shape=[jax.ShapeDtypeStruct((G, H), F32),
               jax.ShapeDtypeStruct((NP, H), F32),
               jax.ShapeDtypeStruct((NP, H), F32),
               jax.ShapeDtypeStruct((NP, H), F32)],
)

_LASTB = pl.pallas_call(
    _lastB_body,
    grid=(NB,),
    in_specs=[_rowblk((RB, H)), _rowblk((RB, G)), _colblk((G, RB)),
              _full((G, H)), _full((G, H)), _full((G, H)), _full((1, H)),
              _full((1, H))],
    out_specs=_full((G, H)),
    out_shape=jax.ShapeDtypeStruct((G, H), F32),
)


def _edge_body(tbl, srcv, dstv, eav, t16, out,
               srcbA, dstbA, eabufA, gbufA,
               srcbB, dstbB, eabufB, gbufB,
               tb, rbuf, acc,
               semA, semEA, semB, semEB):
    c = lax.axis_index("c")
    s = lax.axis_index("s")

    pltpu.sync_copy(t16, tb)

    zero16 = jnp.zeros((16,), F32)

    def zrow(j, carry):
        for v in range(8):
            rbuf[j, pl.ds(v * 16, 16)] = zero16
        return carry

    lax.fori_loop(0, RCH, zrow, 0)
    r0 = s * RPT
    for k in range(RPT // RCH):
        pltpu.sync_copy(rbuf, acc.at[pl.ds(r0 + k * RCH, RCH)])
    plsc.subcore_barrier()

    tv = tb[...]
    ebase = s * EPT
    emax = E - C

    def stage(q, srcb, dstb, eabuf, gbuf, semG, semE):
        base = jnp.minimum(ebase + q * C, emax)
        pltpu.sync_copy(srcv.at[pl.ds(base, C)], srcb)
        pltpu.sync_copy(dstv.at[pl.ds(base, C)], dstb)
        pltpu.async_copy(eav.at[c, pl.ds(base, C)], eabuf, semE)
        pltpu.async_copy(tbl.at[srcb], gbuf, semG)

    def waitbuf(srcb, eabuf, gbuf, semG, semE):
        pltpu.make_async_copy(tbl.at[srcb], gbuf, semG).wait()
        pltpu.make_async_copy(eav.at[c, pl.ds(0, C)], eabuf, semE).wait()

    def compute_scatter(eabuf, gbuf, dstb):
        def compute(off):
            def edge(e, icarry):
                for v in range(4):
                    g = gbuf[e, pl.ds(off + v * 16, 16)]
                    m = jnp.maximum(g + eabuf[e, pl.ds(v * 16, 16)],
                                    0.0) + 1e-7
                    ex = jnp.exp(m * tv)
                    gbuf[e, pl.ds(v * 16, 16)] = ex
                    gbuf[e, pl.ds(64 + v * 16, 16)] = m * ex
                return icarry

            lax.fori_loop(0, C, edge, 0)

        @pl.when(c == 0)
        def _():
            compute(0)

        @pl.when(c == 1)
        def _():
            compute(64)

        pltpu.sync_copy(gbuf, acc.at[dstb], add=True)

    stage(0, srcbA, dstbA, eabufA, gbufA, semA, semEA)

    def body(j, carry):
        i = 2 * j
        stage(i + 1, srcbB, dstbB, eabufB, gbufB, semB, semEB)
        waitbuf(srcbA, eabufA, gbufA, semA, semEA)
        compute_scatter(eabufA, gbufA, dstbA)
        stage(i + 2, srcbA, dstbA, eabufA, gbufA, semA, semEA)
        waitbuf(srcbB, eabufB, gbufB, semB, semEB)
        compute_scatter(eabufB, gbufB, dstbB)
        return carry

    lax.fori_loop(0, NCHUNK // 2, body, 0)
    waitbuf(srcbA, eabufA, gbufA, semA, semEA)
    plsc.subcore_barrier()

    pltpu.sync_copy(acc.at[pl.ds(r0, RPT)], out.at[c, pl.ds(r0, RPT)])


_EDGE = pl.kernel(
    _edge_body,
    out_type=jax.ShapeDtypeStruct((NSC, NP, H), F32),
    mesh=plsc.VectorSubcoreMesh(core_axis_name="c", subcore_axis_name="s",
                                num_cores=NSC, num_subcores=NT),
    scratch_types=[
        pltpu.VMEM((C,), jnp.int32),
        pltpu.VMEM((C,), jnp.int32),
        pltpu.VMEM((C, 64), F32),
        pltpu.VMEM((C, H), F32),
        pltpu.VMEM((C,), jnp.int32),
        pltpu.VMEM((C,), jnp.int32),
        pltpu.VMEM((C, 64), F32),
        pltpu.VMEM((C, H), F32),
        pltpu.VMEM((16,), F32),
        pltpu.VMEM((RCH, H), F32),
        pltpu.VMEM_SHARED((NP, H), F32),
        pltpu.SemaphoreType.DMA,
        pltpu.SemaphoreType.DMA,
        pltpu.SemaphoreType.DMA,
        pltpu.SemaphoreType.DMA,
    ],
)


def kernel(x, edge_index, edge_attr, batch, Wn, bn, We, be, vn_emb, ln_scale,
           ln_bias, W1, b1, mls, mlb, W2, b2, t):
    row = lambda v: v.reshape(1, -1).astype(F32)
    src = edge_index[0].astype(jnp.int32)
    dst = edge_index[1].astype(jnp.int32)
    eattr = jnp.pad(edge_attr.astype(F32), ((0, 0), (0, 8 - 3)))
    wep = jnp.pad(We.astype(F32), ((0, 8 - 3), (0, 0)))
    ea = _EA(eattr, wep)

    onehot = (batch[:, None] == jnp.arange(G, dtype=batch.dtype)[None, :])
    onehot = onehot.astype(F32)
    onehot_p = jnp.pad(onehot, ((0, NP - N), (0, 0)))
    onehotT = onehot_p.T
    xp = jnp.pad(x.astype(F32), ((0, NP - N), (0, 16 - 9)))
    wnp = jnp.pad(Wn.astype(F32), ((0, 16 - 9), (0, 0)))

    h0, hn1, hm, cnt = _PRE(
        xp, wnp, row(bn), row(ln_scale[1]), row(ln_bias[1]), row(be), onehotT)
    vn = jnp.tile(vn_emb.astype(F32), (G, 1))

    hcur = h0
    hn = hn1
    for li in (1, 2, 3):
        t16 = jnp.broadcast_to(t[li].astype(F32), (16,))
        eacc = _EDGE(hm, src, dst, ea, t16)
        hc2, num = _MIDA(hcur, hn, eacc, onehotT, W1[li].astype(F32),
                         row(b1[li]), row(mls[li]), row(mlb[li]),
                         W2[li].astype(F32), row(b2[li]))
        if li < 3:
            vn, hcur, hn, hm = _MIDB(
                hc2, onehot_p, num, cnt, vn, row(ln_scale[li + 1]),
                row(ln_bias[li + 1]), row(be))
        else:
            out = _LASTB(hc2, onehot_p, onehotT, num, cnt, vn,
                         row(ln_scale[0]), row(ln_bias[0]))
    return out


# speedup vs baseline: 1.0258x; 1.0258x over previous
import functools

import jax
import jax.numpy as jnp
from jax import lax
from jax.experimental import pallas as pl
from jax.experimental.pallas import tpu as pltpu
from jax.experimental.pallas import tpu_sc as plsc

N = 10000
E = 320000
H = 128
G = 128

NP = 10240
RB = 2560
NB = NP // RB

NSC = 2
NT = 16
C = 80
EPT = E // NT
NCHUNK = EPT // C
RPT = NP // NT
RCH = 32
F32 = jnp.float32


def _ln(v, scale, bias, eps=1e-5):
    mu = jnp.mean(v, axis=-1, keepdims=True)
    var = jnp.var(v, axis=-1, keepdims=True)
    return (v - mu) / jnp.sqrt(var + eps) * scale + bias


def _pre_body(xp_ref, wn_ref, bn_ref, lns_ref, lnb_ref, be_ref, ohT_ref,
              h_ref, hn_ref, hm_ref, cnt_ref):
    b = pl.program_id(0)
    h = jnp.dot(xp_ref[...], wn_ref[...], preferred_element_type=F32) + bn_ref[...]
    h_ref[...] = h
    hn = jnp.maximum(_ln(h, lns_ref[...], lnb_ref[...]), 0.0)
    hn_ref[...] = hn
    hm_ref[...] = hn + be_ref[...]

    @pl.when(b == 0)
    def _():
        cnt_ref[...] = jnp.zeros_like(cnt_ref)

    cnt_ref[...] += jnp.dot(ohT_ref[...], jnp.ones((RB, G), F32),
                            preferred_element_type=F32)

    @pl.when(b == NB - 1)
    def _():
        cnt_ref[...] = jnp.maximum(cnt_ref[...], 1.0)


def _midA_body(hcur_ref, hn_ref, acc_ref, ohT_ref,
               w1_ref, b1_ref, mls_ref, mlb_ref, w2_ref, b2_ref,
               hc2_ref, num_ref):
    b = pl.program_id(0)
    a = acc_ref[...]
    slabs = []
    for c in range(NSC):
        ex = a[c, :, 0:64]
        mex = a[c, :, 64:128]
        slabs.append(mex / jnp.maximum(ex, 1e-16) + 1e-7)
    out = hn_ref[...] + jnp.concatenate(slabs, axis=1)
    y = jnp.dot(out, w1_ref[...], preferred_element_type=F32) + b1_ref[...]
    y = jnp.maximum(_ln(y, mls_ref[...], mlb_ref[...]), 0.0)
    y = jnp.dot(y, w2_ref[...], preferred_element_type=F32) + b2_ref[...]
    hc2 = hcur_ref[...] + y
    hc2_ref[...] = hc2

    @pl.when(b == 0)
    def _():
        num_ref[...] = jnp.zeros_like(num_ref)

    num_ref[...] += jnp.dot(ohT_ref[...], hc2, preferred_element_type=F32)


def _midB_body(hc2_ref, oh_ref, num_ref, cnt_ref, vnp_ref, lns_ref, lnb_ref,
               be_ref, vn_ref, hcn_ref, hnn_ref, hm_ref):
    vn_new = vnp_ref[...] + num_ref[...] / cnt_ref[...]
    vn_ref[...] = vn_new
    hcn = hc2_ref[...] + jnp.dot(oh_ref[...], vn_new, preferred_element_type=F32)
    hcn_ref[...] = hcn
    z = jnp.maximum(_ln(hcn, lns_ref[...], lnb_ref[...]), 0.0)
    hnn_ref[...] = z
    hm_ref[...] = z + be_ref[...]


def _lastB_body(hc2_ref, oh_ref, ohT_ref, num_ref, cnt_ref, vnp_ref,
                lns_ref, lnb_ref, out_ref):
    b = pl.program_id(0)
    vn_new = vnp_ref[...] + num_ref[...] / cnt_ref[...]
    hcn = hc2_ref[...] + jnp.dot(oh_ref[...], vn_new, preferred_element_type=F32)
    hf = jnp.maximum(_ln(hcn, lns_ref[...], lnb_ref[...]), 0.0)

    @pl.when(b == 0)
    def _():
        out_ref[...] = jnp.zeros_like(out_ref)

    out_ref[...] += jnp.dot(ohT_ref[...], hf, preferred_element_type=F32)

    @pl.when(b == NB - 1)
    def _():
        out_ref[...] = out_ref[...] / cnt_ref[...]


EB = 8000


def _ea_body(eattr_ref, we_ref, o_ref):
    y = jnp.dot(eattr_ref[...], we_ref[...], preferred_element_type=F32)
    o_ref[0] = y[:, 0:64]
    o_ref[1] = y[:, 64:128]


def _rowblk(shape):
    return pl.BlockSpec(shape, lambda b: (b, 0))


def _colblk(shape):
    return pl.BlockSpec(shape, lambda b: (0, b))


def _full(shape):
    return pl.BlockSpec(shape, lambda b: (0, 0))


def _scblk(shape):
    return pl.BlockSpec(shape, lambda b: (0, b, 0))


_EA = pl.pallas_call(
    _ea_body,
    grid=(E // EB,),
    in_specs=[_rowblk((EB, 8)), _full((8, H))],
    out_specs=pl.BlockSpec((2, EB, 64), lambda b: (0, b, 0)),
    out_shape=jax.ShapeDtypeStruct((2, E, 64), F32),
)

_PRE = pl.pallas_call(
    _pre_body,
    grid=(NB,),
    in_specs=[_rowblk((RB, 16)), _full((16, H)), _full((1, H)), _full((1, H)),
              _full((1, H)), _full((1, H)), _colblk((G, RB))],
    out_specs=[_rowblk((RB, H)), _rowblk((RB, H)), _rowblk((RB, H)),
               _full((G, G))],
    out_shape=[jax.ShapeDtypeStruct((NP, H), F32),
               jax.ShapeDtypeStruct((NP, H), F32),
               jax.ShapeDtypeStruct((NP, H), F32),
               jax.ShapeDtypeStruct((G, G), F32)],
)

_MIDA = pl.pallas_call(
    _midA_body,
    grid=(NB,),
    in_specs=[_rowblk((RB, H)), _rowblk((RB, H)),
              _scblk((NSC, RB, H)),
              _colblk((G, RB)), _full((H, 2 * H)),
              _full((1, 2 * H)), _full((1, 2 * H)), _full((1, 2 * H)),
              _full((2 * H, H)), _full((1, H))],
    out_specs=[_rowblk((RB, H)), _full((G, H))],
    out_shape=[jax.ShapeDtypeStruct((NP, H), F32),
               jax.ShapeDtypeStruct((G, H), F32)],
)

_MIDB = pl.pallas_call(
    _midB_body,
    grid=(NB,),
    in_specs=[_rowblk((RB, H)), _rowblk((RB, G)), _full((G, H)), _full((G, H)),
              _full((G, H)), _full((1, H)), _full((1, H)), _full((1, H))],
    out_specs=[_full((G, H)), _rowblk((RB, H)), _rowblk((RB, H)),
               _rowblk((RB, H))],
    out_shape=[jax.ShapeDtypeStruct((G, H), F32),
               jax.ShapeDtypeStruct((NP, H), F32),
               jax.ShapeDtypeStruct((NP, H), F32),
               jax.ShapeDtypeStruct((NP, H), F32)],
)

_LASTB = pl.pallas_call(
    _lastB_body,
    grid=(NB,),
    in_specs=[_rowblk((RB, H)), _rowblk((RB, G)), _colblk((G, RB)),
              _full((G, H)), _full((G, H)), _full((G, H)), _full((1, H)),
              _full((1, H))],
    out_specs=_full((G, H)),
    out_shape=jax.ShapeDtypeStruct((G, H), F32),
)


def _edge_body(tbl, srcv, dstv, eav, t16, out,
               srcbA, dstbA, eabufA, gbufA,
               srcbB, dstbB, eabufB, gbufB,
               tb, rbuf, acc,
               semA, semEA, semB, semEB):
    c = lax.axis_index("c")
    s = lax.axis_index("s")

    pltpu.sync_copy(t16, tb)

    zero16 = jnp.zeros((16,), F32)

    def zrow(j, carry):
        for v in range(8):
            rbuf[j, pl.ds(v * 16, 16)] = zero16
        return carry

    lax.fori_loop(0, RCH, zrow, 0)
    r0 = s * RPT
    for k in range(RPT // RCH):
        pltpu.sync_copy(rbuf, acc.at[pl.ds(r0 + k * RCH, RCH)])
    plsc.subcore_barrier()

    tv = tb[...]
    ebase = s * EPT
    emax = E - C

    def stage(q, srcb, dstb, eabuf, gbuf, semG, semE):
        base = jnp.minimum(ebase + q * C, emax)
        pltpu.sync_copy(srcv.at[pl.ds(base, C)], srcb)
        pltpu.sync_copy(dstv.at[pl.ds(base, C)], dstb)
        pltpu.async_copy(eav.at[c, pl.ds(base, C)], eabuf, semE)
        pltpu.async_copy(tbl.at[srcb], gbuf, semG)

    def waitbuf(srcb, eabuf, gbuf, semG, semE):
        pltpu.make_async_copy(tbl.at[srcb], gbuf, semG).wait()
        pltpu.make_async_copy(eav.at[c, pl.ds(0, C)], eabuf, semE).wait()

    def compute_scatter(eabuf, gbuf, dstb):
        def compute(off):
            def edge(e, icarry):
                for v in range(4):
                    g = gbuf[e, pl.ds(off + v * 16, 16)]
                    m = jnp.maximum(g + eabuf[e, pl.ds(v * 16, 16)], 0.0)
                    ex = jnp.exp(m * tv)
                    gbuf[e, pl.ds(v * 16, 16)] = ex
                    gbuf[e, pl.ds(64 + v * 16, 16)] = m * ex
                return icarry

            lax.fori_loop(0, C, edge, 0)

        @pl.when(c == 0)
        def _():
            compute(0)

        @pl.when(c == 1)
        def _():
            compute(64)

        pltpu.sync_copy(gbuf, acc.at[dstb], add=True)

    stage(0, srcbA, dstbA, eabufA, gbufA, semA, semEA)

    def body(j, carry):
        i = 2 * j
        stage(i + 1, srcbB, dstbB, eabufB, gbufB, semB, semEB)
        waitbuf(srcbA, eabufA, gbufA, semA, semEA)
        compute_scatter(eabufA, gbufA, dstbA)
        stage(i + 2, srcbA, dstbA, eabufA, gbufA, semA, semEA)
        waitbuf(srcbB, eabufB, gbufB, semB, semEB)
        compute_scatter(eabufB, gbufB, dstbB)
        return carry

    lax.fori_loop(0, NCHUNK // 2, body, 0)
    waitbuf(srcbA, eabufA, gbufA, semA, semEA)
    plsc.subcore_barrier()

    pltpu.sync_copy(acc.at[pl.ds(r0, RPT)], out.at[c, pl.ds(r0, RPT)])


_EDGE = pl.kernel(
    _edge_body,
    out_type=jax.ShapeDtypeStruct((NSC, NP, H), F32),
    mesh=plsc.VectorSubcoreMesh(core_axis_name="c", subcore_axis_name="s",
                                num_cores=NSC, num_subcores=NT),
    scratch_types=[
        pltpu.VMEM((C,), jnp.int32),
        pltpu.VMEM((C,), jnp.int32),
        pltpu.VMEM((C, 64), F32),
        pltpu.VMEM((C, H), F32),
        pltpu.VMEM((C,), jnp.int32),
        pltpu.VMEM((C,), jnp.int32),
        pltpu.VMEM((C, 64), F32),
        pltpu.VMEM((C, H), F32),
        pltpu.VMEM((16,), F32),
        pltpu.VMEM((RCH, H), F32),
        pltpu.VMEM_SHARED((NP, H), F32),
        pltpu.SemaphoreType.DMA,
        pltpu.SemaphoreType.DMA,
        pltpu.SemaphoreType.DMA,
        pltpu.SemaphoreType.DMA,
    ],
)


def kernel(x, edge_index, edge_attr, batch, Wn, bn, We, be, vn_emb, ln_scale,
           ln_bias, W1, b1, mls, mlb, W2, b2, t):
    row = lambda v: v.reshape(1, -1).astype(F32)
    src = edge_index[0].astype(jnp.int32)
    dst = edge_index[1].astype(jnp.int32)
    eattr = jnp.pad(edge_attr.astype(F32), ((0, 0), (0, 8 - 3)))
    wep = jnp.pad(We.astype(F32), ((0, 8 - 3), (0, 0)))
    ea = _EA(eattr, wep)

    onehot = (batch[:, None] == jnp.arange(G, dtype=batch.dtype)[None, :])
    onehot = onehot.astype(F32)
    onehot_p = jnp.pad(onehot, ((0, NP - N), (0, 0)))
    onehotT = onehot_p.T
    xp = jnp.pad(x.astype(F32), ((0, NP - N), (0, 16 - 9)))
    wnp = jnp.pad(Wn.astype(F32), ((0, 16 - 9), (0, 0)))

    h0, hn1, hm, cnt = _PRE(
        xp, wnp, row(bn), row(ln_scale[1]), row(ln_bias[1]), row(be), onehotT)
    vn = jnp.tile(vn_emb.astype(F32), (G, 1))

    hcur = h0
    hn = hn1
    for li in (1, 2, 3):
        t16 = jnp.broadcast_to(t[li].astype(F32), (16,))
        eacc = _EDGE(hm, src, dst, ea, t16)
        hc2, num = _MIDA(hcur, hn, eacc, onehotT, W1[li].astype(F32),
                         row(b1[li]), row(mls[li]), row(mlb[li]),
                         W2[li].astype(F32), row(b2[li]))
        if li < 3:
            vn, hcur, hn, hm = _MIDB(
                hc2, onehot_p, num, cnt, vn, row(ln_scale[li + 1]),
                row(ln_bias[li + 1]), row(be))
        else:
            out = _LASTB(hc2, onehot_p, onehotT, num, cnt, vn,
                         row(ln_scale[0]), row(ln_bias[0]))
    return out
